# Initial kernel scaffold; baseline (speedup 1.0000x reference)
import jax
import jax.numpy as jnp
from jax.experimental import pallas as pl


def kernel(x, edge_index, W1, b1, W2, b2):
    raise NotImplementedError("write your pallas kernel here")


# speedup vs baseline: 8.4962x; 8.4962x over previous
import functools

import jax
import jax.numpy as jnp
from jax import lax
from jax.experimental import pallas as pl
from jax.experimental.pallas import tpu as pltpu
from jax.experimental.pallas import tpu_sc as plsc

N = 10000
NPAD = 10240
E = 320000
F = 128
H = 128
C = 40
CP = 128
NC, NS = 2, 16
NW = NC * NS
CHUNK = 128
CPW = 80
EPAD = NW * CPW * CHUNK
STRIPE = NPAD // NS

_mesh = plsc.VectorSubcoreMesh(core_axis_name="c", subcore_axis_name="s")


@functools.partial(
    pl.kernel,
    out_type=jax.ShapeDtypeStruct((NC, NPAD, 128), jnp.float32),
    mesh=_mesh,
    scratch_types=[
        pltpu.VMEM_SHARED((NPAD, 128), jnp.float32),
        pltpu.VMEM((CPW, CHUNK), jnp.int32),
        pltpu.VMEM((CHUNK, 128), jnp.float32),
        pltpu.VMEM((CHUNK, 128), jnp.float32),
    ],
)
def _sc_degree(col_hbm, ones_hbm, z_hbm, out_hbm, acc, colv, onesv, tmpv):
    c = lax.axis_index("c")
    s = lax.axis_index("s")
    w = c * NS + s
    base = s * STRIPE
    pltpu.sync_copy(z_hbm, tmpv)
    for t in range(STRIPE // CHUNK):
        pltpu.sync_copy(tmpv, acc.at[pl.ds(base + t * CHUNK, CHUNK)])
    pltpu.sync_copy(ones_hbm, onesv)
    pltpu.sync_copy(col_hbm.at[w], colv)
    plsc.subcore_barrier()

    @pl.loop(0, CPW)
    def _(j):
        pltpu.sync_copy(onesv, acc.at[colv.at[j]], add=True)

    plsc.subcore_barrier()
    for t in range(STRIPE // CHUNK):
        pltpu.sync_copy(acc.at[pl.ds(base + t * CHUNK, CHUNK)], tmpv)
        pltpu.sync_copy(tmpv, out_hbm.at[c, pl.ds(base + t * CHUNK, CHUNK)])


def _make_prop(d, nep):
    cpe = CPW // nep

    @functools.partial(
        pl.kernel,
        out_type=jax.ShapeDtypeStruct((NC, NPAD, d), jnp.float32),
        mesh=_mesh,
        scratch_types=[
            pltpu.VMEM_SHARED((NPAD, d), jnp.float32),
            pltpu.VMEM((cpe, CHUNK), jnp.int32),
            pltpu.VMEM((cpe, CHUNK), jnp.int32),
            pltpu.VMEM((CHUNK, d), jnp.float32),
            pltpu.VMEM((CHUNK, d), jnp.float32),
            pltpu.SemaphoreType.DMA,
            pltpu.SemaphoreType.DMA,
        ],
    )
    def prop(hs_hbm, row_hbm, col_hbm, z_hbm, out_hbm,
             acc, rowv, colv, buf0, buf1, sem0, sem1):
        c = lax.axis_index("c")
        s = lax.axis_index("s")
        w = c * NS + s
        base = s * STRIPE
        pltpu.sync_copy(z_hbm, buf0)
        for t in range(STRIPE // CHUNK):
            pltpu.sync_copy(buf0, acc.at[pl.ds(base + t * CHUNK, CHUNK)])
        plsc.subcore_barrier()

        for ep in range(nep):
            pltpu.sync_copy(row_hbm.at[w, pl.ds(ep * cpe, cpe)], rowv)
            pltpu.sync_copy(col_hbm.at[w, pl.ds(ep * cpe, cpe)], colv)

            pltpu.async_copy(hs_hbm.at[rowv.at[0]], buf0, sem0)

            @pl.loop(0, cpe // 2)
            def _(g):
                j0 = 2 * g
                a1 = pltpu.async_copy(hs_hbm.at[rowv.at[j0 + 1]], buf1, sem1)
                pltpu.make_async_copy(hs_hbm.at[rowv.at[j0]], buf0, sem0).wait()
                pltpu.sync_copy(buf0, acc.at[colv.at[j0]], add=True)

                @pl.when(g + 1 < cpe // 2)
                def _():
                    pltpu.async_copy(hs_hbm.at[rowv.at[j0 + 2]], buf0, sem0)

                a1.wait()
                pltpu.sync_copy(buf1, acc.at[colv.at[j0 + 1]], add=True)

        plsc.subcore_barrier()
        for t in range(STRIPE // CHUNK):
            pltpu.sync_copy(acc.at[pl.ds(base + t * CHUNK, CHUNK)], buf0)
            pltpu.sync_copy(buf0, out_hbm.at[c, pl.ds(base + t * CHUNK, CHUNK)])

    return prop


_prop_h = _make_prop(H, 2)
_prop_c = _prop_h


def _dis(deg_ref):
    return lax.rsqrt(deg_ref[0, :, 0:1] + deg_ref[1, :, 0:1] + 1.0)


def _tc1_body(x_ref, w_ref, deg_ref, o_ref):
    h = jnp.dot(x_ref[...], w_ref[...], preferred_element_type=jnp.float32)
    o_ref[...] = h * _dis(deg_ref)


def _tc2_body(p_ref, hs1_ref, deg_ref, b1_ref, w2_ref, o_ref):
    dis = _dis(deg_ref)
    p = p_ref[0] + p_ref[1] + hs1_ref[...]
    z = jnp.maximum(p * dis + b1_ref[...], 0.0)
    rowid = lax.broadcasted_iota(jnp.int32, (NPAD, 1), 0)
    disz = jnp.where(rowid < N, dis, 0.0)
    o_ref[...] = jnp.dot(z, w2_ref[...],
                         preferred_element_type=jnp.float32) * disz


def _tc3_body(q_ref, hs2_ref, deg_ref, b2_ref, o_ref):
    p2 = q_ref[0] + q_ref[1] + hs2_ref[...]
    o_ref[...] = p2 * _dis(deg_ref) + b2_ref[...]


def kernel(x, edge_index, W1, b1, W2, b2):
    xp = jnp.pad(x, ((0, NPAD - N), (0, 0)))
    pad = jnp.full((EPAD - E,), N, dtype=jnp.int32)
    row = jnp.concatenate([edge_index[0], pad]).reshape(NW, CPW, CHUNK)
    col = jnp.concatenate([edge_index[1], pad]).reshape(NW, CPW, CHUNK)
    w2p = jnp.pad(W2, ((0, 0), (0, CP - C)))
    b1r = b1.reshape(1, H)
    b2r = jnp.pad(b2, (0, CP - C)).reshape(1, CP)
    ones16 = jnp.ones((CHUNK, 128), jnp.float32)
    zdeg = jnp.zeros((CHUNK, 128), jnp.float32)
    zh = jnp.zeros((CHUNK, H), jnp.float32)
    zc = jnp.zeros((CHUNK, CP), jnp.float32)

    degp = _sc_degree(col, ones16, zdeg)

    hs1 = pl.pallas_call(
        _tc1_body,
        out_shape=jax.ShapeDtypeStruct((NPAD, H), jnp.float32),
    )(xp, W1, degp)

    P = _prop_h(hs1, row, col, zh)

    hs2 = pl.pallas_call(
        _tc2_body,
        out_shape=jax.ShapeDtypeStruct((NPAD, CP), jnp.float32),
    )(P, hs1, degp, b1r, w2p)

    Q = _prop_c(hs2, row, col, zc)

    y = pl.pallas_call(
        _tc3_body,
        out_shape=jax.ShapeDtypeStruct((NPAD, CP), jnp.float32),
    )(Q, hs2, degp, b2r)

    return y[:N, :C]


# speedup vs baseline: 26.3688x; 3.1036x over previous
import functools

import jax
import jax.numpy as jnp
from jax import lax
from jax.experimental import pallas as pl
from jax.experimental.pallas import tpu as pltpu
from jax.experimental.pallas import tpu_sc as plsc

N = 10000
NPAD = 10240
E = 320000
F = 128
H = 128
C = 40
CP = 128
NC, NS = 2, 16
NW = NC * NS
CHUNK = 128
CPW = 80
EPAD = NW * CPW * CHUNK
STRIPE = NPAD // NS

_mesh = plsc.VectorSubcoreMesh(core_axis_name="c", subcore_axis_name="s")


@functools.partial(
    pl.kernel,
    out_type=jax.ShapeDtypeStruct((NC, NPAD, 128), jnp.float32),
    mesh=_mesh,
    scratch_types=[
        pltpu.VMEM_SHARED((NPAD, 128), jnp.float32),
        pltpu.VMEM((CPW, CHUNK), jnp.int32),
        pltpu.VMEM((CHUNK, 128), jnp.float32),
        pltpu.VMEM((CHUNK, 128), jnp.float32),
    ],
)
def _sc_degree(col_hbm, ones_hbm, z_hbm, out_hbm, acc, colv, onesv, tmpv):
    c = lax.axis_index("c")
    s = lax.axis_index("s")
    w = c * NS + s
    base = s * STRIPE
    pltpu.sync_copy(z_hbm, tmpv)
    for t in range(STRIPE // CHUNK):
        pltpu.sync_copy(tmpv, acc.at[pl.ds(base + t * CHUNK, CHUNK)])
    pltpu.sync_copy(ones_hbm, onesv)
    pltpu.sync_copy(col_hbm.at[w], colv)
    plsc.subcore_barrier()

    @pl.loop(0, CPW)
    def _(j):
        pltpu.sync_copy(onesv, acc.at[colv.at[j]], add=True)

    plsc.subcore_barrier()
    for t in range(STRIPE // CHUNK):
        pltpu.sync_copy(acc.at[pl.ds(base + t * CHUNK, CHUNK)], tmpv)
        pltpu.sync_copy(tmpv, out_hbm.at[c, pl.ds(base + t * CHUNK, CHUNK)])


def _make_prop(d, nep):
    cpe = CPW // nep

    @functools.partial(
        pl.kernel,
        out_type=jax.ShapeDtypeStruct((NC, NPAD, d), jnp.float32),
        mesh=_mesh,
        scratch_types=[
            pltpu.VMEM_SHARED((NPAD, d), jnp.float32),
            pltpu.VMEM((cpe, CHUNK), jnp.int32),
            pltpu.VMEM((cpe, CHUNK), jnp.int32),
            pltpu.VMEM((CHUNK, d), jnp.float32),
            pltpu.VMEM((CHUNK, d), jnp.float32),
            pltpu.SemaphoreType.DMA,
            pltpu.SemaphoreType.DMA,
        ],
    )
    def prop(hs_hbm, row_hbm, col_hbm, z_hbm, out_hbm,
             acc, rowv, colv, buf0, buf1, sem0, sem1):
        c = lax.axis_index("c")
        s = lax.axis_index("s")
        w = c * NS + s
        base = s * STRIPE
        pltpu.sync_copy(z_hbm, buf0)
        for t in range(STRIPE // CHUNK):
            pltpu.sync_copy(buf0, acc.at[pl.ds(base + t * CHUNK, CHUNK)])
        plsc.subcore_barrier()

        for ep in range(nep):
            pltpu.sync_copy(row_hbm.at[w, pl.ds(ep * cpe, cpe)], rowv)
            pltpu.sync_copy(col_hbm.at[w, pl.ds(ep * cpe, cpe)], colv)

            pltpu.async_copy(hs_hbm.at[rowv.at[0]], buf0, sem0)

            @pl.loop(0, cpe // 2)
            def _(g):
                j0 = 2 * g
                a1 = pltpu.async_copy(hs_hbm.at[rowv.at[j0 + 1]], buf1, sem1)
                pltpu.make_async_copy(hs_hbm.at[rowv.at[j0]], buf0, sem0).wait()
                pltpu.sync_copy(buf0, acc.at[colv.at[j0]], add=True)

                @pl.when(g + 1 < cpe // 2)
                def _():
                    pltpu.async_copy(hs_hbm.at[rowv.at[j0 + 2]], buf0, sem0)

                a1.wait()
                pltpu.sync_copy(buf1, acc.at[colv.at[j0 + 1]], add=True)

        plsc.subcore_barrier()
        for t in range(STRIPE // CHUNK):
            pltpu.sync_copy(acc.at[pl.ds(base + t * CHUNK, CHUNK)], buf0)
            pltpu.sync_copy(buf0, out_hbm.at[c, pl.ds(base + t * CHUNK, CHUNK)])

    return prop


_prop_h = _make_prop(H, 2)
_prop_c = _prop_h


def _dis(deg_ref):
    return lax.rsqrt(deg_ref[0, :, 0:1] + deg_ref[1, :, 0:1] + 1.0)


def _tc1_body(x_ref, w_ref, deg_ref, o_ref):
    h = jnp.dot(x_ref[...], w_ref[...], preferred_element_type=jnp.float32)
    o_ref[...] = h * _dis(deg_ref)


def _tc2_body(p_ref, hs1_ref, deg_ref, b1_ref, w2_ref, o_ref):
    dis = _dis(deg_ref)
    p = p_ref[0] + p_ref[1] + hs1_ref[...]
    z = jnp.maximum(p * dis + b1_ref[...], 0.0)
    rowid = lax.broadcasted_iota(jnp.int32, (NPAD, 1), 0)
    disz = jnp.where(rowid < N, dis, 0.0)
    o_ref[...] = jnp.dot(z, w2_ref[...],
                         preferred_element_type=jnp.float32) * disz


def _tc3_body(q_ref, hs2_ref, deg_ref, b2_ref, o_ref):
    p2 = q_ref[0] + q_ref[1] + hs2_ref[...]
    o_ref[...] = p2 * _dis(deg_ref) + b2_ref[...]


def kernel(x, edge_index, W1, b1, W2, b2):
    xp = jnp.pad(x, ((0, NPAD - N), (0, 0)))
    epw = E // NW
    padw = (EPAD - E) // NW
    padv = jnp.broadcast_to(
        N + (jnp.arange(padw, dtype=jnp.int32) % (NPAD - N)), (NW, padw))
    row = jnp.concatenate(
        [edge_index[0].reshape(NW, epw), padv], axis=1).reshape(NW, CPW, CHUNK)
    col = jnp.concatenate(
        [edge_index[1].reshape(NW, epw), padv], axis=1).reshape(NW, CPW, CHUNK)
    w2p = jnp.pad(W2, ((0, 0), (0, CP - C)))
    b1r = b1.reshape(1, H)
    b2r = jnp.pad(b2, (0, CP - C)).reshape(1, CP)
    ones16 = jnp.ones((CHUNK, 128), jnp.float32)
    zdeg = jnp.zeros((CHUNK, 128), jnp.float32)
    zh = jnp.zeros((CHUNK, H), jnp.float32)
    zc = jnp.zeros((CHUNK, CP), jnp.float32)

    degp = _sc_degree(col, ones16, zdeg)

    hs1 = pl.pallas_call(
        _tc1_body,
        out_shape=jax.ShapeDtypeStruct((NPAD, H), jnp.float32),
    )(xp, W1, degp)

    P = _prop_h(hs1, row, col, zh)

    hs2 = pl.pallas_call(
        _tc2_body,
        out_shape=jax.ShapeDtypeStruct((NPAD, CP), jnp.float32),
    )(P, hs1, degp, b1r, w2p)

    Q = _prop_c(hs2, row, col, zc)

    y = pl.pallas_call(
        _tc3_body,
        out_shape=jax.ShapeDtypeStruct((NPAD, CP), jnp.float32),
    )(Q, hs2, degp, b2r)

    return y[:N, :C]


# speedup vs baseline: 36.2942x; 1.3764x over previous
import functools

import jax
import jax.numpy as jnp
from jax import lax
from jax.experimental import pallas as pl
from jax.experimental.pallas import tpu as pltpu
from jax.experimental.pallas import tpu_sc as plsc

N = 10000
NPAD = 10240
E = 320000
F = 128
H = 128
C = 40
CP = 48
NC, NS = 2, 16
NW = NC * NS
CHUNK = 128
CPW = 80
EPAD = NW * CPW * CHUNK
STRIPE = NPAD // NS

_mesh = plsc.VectorSubcoreMesh(core_axis_name="c", subcore_axis_name="s")


@functools.partial(
    pl.kernel,
    out_type=jax.ShapeDtypeStruct((NW, NPAD), jnp.float32),
    mesh=_mesh,
    scratch_types=[
        pltpu.VMEM((NPAD,), jnp.float32),
        pltpu.VMEM((CPW, CHUNK), jnp.int32),
    ],
    compiler_params=pltpu.CompilerParams(needs_layout_passes=False),
)
def _sc_degree(col_hbm, z_hbm, out_hbm, hist, colv):
    c = lax.axis_index("c")
    s = lax.axis_index("s")
    w = c * NS + s
    pltpu.sync_copy(z_hbm, hist)
    pltpu.sync_copy(col_hbm.at[w], colv)
    ones = jnp.full((16,), 1.0, jnp.float32)

    @pl.loop(0, CPW)
    def _(j):
        for k in range(CHUNK // 16):
            idx = colv[j, pl.ds(k * 16, 16)]
            plsc.addupdate_scatter(hist, [idx], ones)

    pltpu.sync_copy(hist, out_hbm.at[w])


def _make_prop(d, nep, tc_tiling=True):
    cpe = CPW // nep

    @functools.partial(
        pl.kernel,
        out_type=jax.ShapeDtypeStruct((NC, NPAD, d), jnp.float32),
        mesh=_mesh,
        scratch_types=[
            pltpu.VMEM_SHARED((NPAD, d), jnp.float32),
            pltpu.VMEM((cpe, CHUNK), jnp.int32),
            pltpu.VMEM((cpe, CHUNK), jnp.int32),
            pltpu.VMEM((CHUNK, d), jnp.float32),
            pltpu.VMEM((CHUNK, d), jnp.float32),
            pltpu.SemaphoreType.DMA,
            pltpu.SemaphoreType.DMA,
        ],
        compiler_params=pltpu.CompilerParams(use_tc_tiling_on_sc=tc_tiling),
    )
    def prop(hs_hbm, row_hbm, col_hbm, z_hbm, out_hbm,
             acc, rowv, colv, buf0, buf1, sem0, sem1):
        c = lax.axis_index("c")
        s = lax.axis_index("s")
        w = c * NS + s
        base = s * STRIPE
        pltpu.sync_copy(z_hbm, buf0)
        for t in range(STRIPE // CHUNK):
            pltpu.sync_copy(buf0, acc.at[pl.ds(base + t * CHUNK, CHUNK)])
        plsc.subcore_barrier()

        for ep in range(nep):
            pltpu.sync_copy(row_hbm.at[w, pl.ds(ep * cpe, cpe)], rowv)
            pltpu.sync_copy(col_hbm.at[w, pl.ds(ep * cpe, cpe)], colv)

            pltpu.async_copy(hs_hbm.at[rowv.at[0]], buf0, sem0)

            @pl.loop(0, cpe // 2)
            def _(g):
                j0 = 2 * g
                a1 = pltpu.async_copy(hs_hbm.at[rowv.at[j0 + 1]], buf1, sem1)
                pltpu.make_async_copy(hs_hbm.at[rowv.at[j0]], buf0, sem0).wait()
                pltpu.sync_copy(buf0, acc.at[colv.at[j0]], add=True)

                @pl.when(g + 1 < cpe // 2)
                def _():
                    pltpu.async_copy(hs_hbm.at[rowv.at[j0 + 2]], buf0, sem0)

                a1.wait()
                pltpu.sync_copy(buf1, acc.at[colv.at[j0 + 1]], add=True)

        plsc.subcore_barrier()
        for t in range(STRIPE // CHUNK):
            pltpu.sync_copy(acc.at[pl.ds(base + t * CHUNK, CHUNK)], buf0)
            pltpu.sync_copy(buf0, out_hbm.at[c, pl.ds(base + t * CHUNK, CHUNK)])

    return prop


_prop_h = _make_prop(H, 2)
_prop_c = _make_prop(CP, 1, tc_tiling=False)


def _dis(deg_ref):
    return lax.rsqrt(jnp.sum(deg_ref[...], axis=0)[:, None] + 1.0)


def _tc1_body(x_ref, w_ref, deg_ref, o_ref):
    h = jnp.dot(x_ref[...], w_ref[...], preferred_element_type=jnp.float32)
    o_ref[...] = h * _dis(deg_ref)


def _tc2_body(p_ref, hs1_ref, deg_ref, b1_ref, w2_ref, o_ref):
    dis = _dis(deg_ref)
    p = p_ref[0] + p_ref[1] + hs1_ref[...]
    z = jnp.maximum(p * dis + b1_ref[...], 0.0)
    rowid = lax.broadcasted_iota(jnp.int32, (NPAD, 1), 0)
    disz = jnp.where(rowid < N, dis, 0.0)
    o_ref[...] = jnp.dot(z, w2_ref[...],
                         preferred_element_type=jnp.float32) * disz


def _tc3_body(q_ref, hs2_ref, deg_ref, b2_ref, o_ref):
    p2 = q_ref[0] + q_ref[1] + hs2_ref[...]
    o_ref[...] = p2 * _dis(deg_ref) + b2_ref[...]


def kernel(x, edge_index, W1, b1, W2, b2):
    xp = jnp.pad(x, ((0, NPAD - N), (0, 0)))
    epw = E // NW
    padw = (EPAD - E) // NW
    padv = jnp.broadcast_to(
        N + (jnp.arange(padw, dtype=jnp.int32) % (NPAD - N)), (NW, padw))
    row = jnp.concatenate(
        [edge_index[0].reshape(NW, epw), padv], axis=1).reshape(NW, CPW, CHUNK)
    col = jnp.concatenate(
        [edge_index[1].reshape(NW, epw), padv], axis=1).reshape(NW, CPW, CHUNK)
    w2p = jnp.pad(W2, ((0, 0), (0, CP - C)))
    b1r = b1.reshape(1, H)
    b2r = jnp.pad(b2, (0, CP - C)).reshape(1, CP)
    zdeg = jnp.zeros((NPAD,), jnp.float32)
    zh = jnp.zeros((CHUNK, H), jnp.float32)
    zc = jnp.zeros((CHUNK, CP), jnp.float32)

    degp = _sc_degree(col, zdeg)

    hs1 = pl.pallas_call(
        _tc1_body,
        out_shape=jax.ShapeDtypeStruct((NPAD, H), jnp.float32),
    )(xp, W1, degp)

    P = _prop_h(hs1, row, col, zh)

    hs2 = pl.pallas_call(
        _tc2_body,
        out_shape=jax.ShapeDtypeStruct((NPAD, CP), jnp.float32),
    )(P, hs1, degp, b1r, w2p)

    Q = _prop_c(hs2, row, col, zc)

    y = pl.pallas_call(
        _tc3_body,
        out_shape=jax.ShapeDtypeStruct((NPAD, CP), jnp.float32),
    )(Q, hs2, degp, b2r)

    return y[:N, :C]
